# baseline (speedup 1.0000x reference)
import functools

import jax
import jax.numpy as jnp
from jax import lax
from jax.experimental import pallas as pl
from jax.experimental.pallas import tpu as pltpu
from jax.experimental.pallas import tpu_sc as plsc

H, W, D = 32, 32, 768
B = 32
DH = D // 2
L = 16
NC, NS = 2, 16
HPS = H // NS


def _body(scale_hbm, row_hbm, col_hbm, out_hbm, sbuf, rowbuf, colbuf, stage, sem):
    cid = lax.axis_index("c")
    sid = lax.axis_index("s")
    pltpu.sync_copy(scale_hbm, sbuf)
    pltpu.sync_copy(row_hbm, rowbuf)
    pltpu.sync_copy(col_hbm, colbuf)
    s = sbuf[...]
    h0 = HPS * sid
    for j in range(HPS):
        rchunks = [rowbuf[h0 + j, pl.ds(L * k, L)] * s for k in range(DH // L)]

        def wbody(w, carry, j=j, rchunks=rchunks):
            for k in range(DH // L):
                stage[j, w, pl.ds(L * k, L)] = rchunks[k]
                stage[j, w, pl.ds(DH + L * k, L)] = colbuf[w, pl.ds(L * k, L)] * s
            return carry

        lax.fori_loop(0, W, wbody, 0)
    b0 = cid * NS
    copies = [
        pltpu.async_copy(stage, out_hbm.at[b0 + b, pl.ds(h0, HPS)], sem)
        for b in range(NS)
    ]
    for cp in copies:
        cp.wait()


def kernel(row_embed, col_embed, batch_size):
    scale = (jnp.asarray(batch_size, jnp.int32) // B).astype(jnp.float32)
    scale_vec = jnp.full((L,), scale, dtype=jnp.float32)
    mesh = plsc.VectorSubcoreMesh(core_axis_name="c", subcore_axis_name="s")
    run = functools.partial(
        pl.kernel,
        mesh=mesh,
        out_type=jax.ShapeDtypeStruct((B, H, W, D), jnp.float32),
        scratch_types=[
            pltpu.VMEM((L,), jnp.float32),
            pltpu.VMEM((H, DH), jnp.float32),
            pltpu.VMEM((W, DH), jnp.float32),
            pltpu.VMEM((HPS, W, D), jnp.float32),
            pltpu.SemaphoreType.DMA,
        ],
    )(_body)
    return run(scale_vec, row_embed, col_embed)
